# baseline (speedup 1.0000x reference)
import functools

import jax
import jax.numpy as jnp
from jax import lax
from jax.experimental import pallas as pl
from jax.experimental.pallas import tpu as pltpu
from jax.experimental.pallas import tpu_sc as plsc

_B = 16384
_D = 64
_NC = 2
_NS = 16
_NW = _NC * _NS
_BPW = _B // _NW
_CH = 128
_NCH = _BPW // _CH
_L = 16


def _dot_kernel(uid_hbm, iid_hbm, ut_hbm, it_hbm, out_hbm,
                iu_v, ii_v, ru_v, ri_v, o_v, sem):
    wid = lax.axis_index("s") * _NC + lax.axis_index("c")
    base = wid * _BPW

    for c in range(_NCH):
        pltpu.sync_copy(uid_hbm.at[pl.ds(base + c * _CH, _CH)], iu_v.at[c])
        pltpu.sync_copy(iid_hbm.at[pl.ds(base + c * _CH, _CH)], ii_v.at[c])

    def fire_group(c, tbl, dst):
        def fire_d(d, carry):
            pltpu.async_copy(tbl.at[d].at[iu_v.at[c] if tbl is ut_hbm
                                          else ii_v.at[c]],
                             dst.at[d, pl.ds(c * _CH, _CH)], sem)
            return carry
        lax.fori_loop(0, _D, fire_d, 0)

    def drain_group(c, tbl, dst):
        pltpu.make_async_copy(tbl.at[:, pl.ds(0, _CH)],
                              dst.at[:, pl.ds(c * _CH, _CH)], sem).wait()

    groups = []
    for c in range(_NCH):
        groups.append((c, ut_hbm, ru_v))
        groups.append((c, it_hbm, ri_v))
    fire_group(*groups[0])
    for k in range(1, len(groups)):
        fire_group(*groups[k])
        drain_group(*groups[k - 1])
    drain_group(*groups[-1])

    def body(g, carry):
        col = g * _L
        acc = ru_v[0, pl.ds(col, _L)] * ri_v[0, pl.ds(col, _L)]
        for d in range(1, _D):
            acc = acc + ru_v[d, pl.ds(col, _L)] * ri_v[d, pl.ds(col, _L)]
        o_v[pl.ds(col, _L)] = acc
        return carry

    lax.fori_loop(0, _BPW // _L, body, 0)

    pltpu.sync_copy(o_v, out_hbm.at[pl.ds(base, _BPW)])


@jax.jit
def kernel(user_ids, item_ids, user_table, item_table):
    run = functools.partial(
        pl.kernel,
        mesh=plsc.VectorSubcoreMesh(core_axis_name="c", subcore_axis_name="s"),
        out_type=jax.ShapeDtypeStruct((_B,), jnp.float32),
        scratch_types=[
            pltpu.VMEM((_NCH, _CH), jnp.int32),
            pltpu.VMEM((_NCH, _CH), jnp.int32),
            pltpu.VMEM((_D, _BPW), jnp.float32),
            pltpu.VMEM((_D, _BPW), jnp.float32),
            pltpu.VMEM((_BPW,), jnp.float32),
            pltpu.SemaphoreType.DMA,
        ],
        compiler_params=pltpu.CompilerParams(use_tc_tiling_on_sc=False),
    )(_dot_kernel)
    out = run(user_ids, item_ids, user_table.T, item_table.T)
    return out.reshape(_B, 1)


# speedup vs baseline: 9.1356x; 9.1356x over previous
import functools

import jax
import jax.numpy as jnp
from jax import lax
from jax.experimental import pallas as pl
from jax.experimental.pallas import tpu as pltpu
from jax.experimental.pallas import tpu_sc as plsc

_B = 16384
_D = 64
_NC = 2
_NS = 16
_NW = _NC * _NS
_BPW = _B // _NW
_CH = 128
_NCH = _BPW // _CH
_L = 16
_PAIRS = 500000


def _dot_kernel(uid_hbm, iid_hbm, ut_hbm, it_hbm, out_hbm,
                idu_v, idi_v, ju_v, ji_v, pu_v, pi_v, o_v, sem):
    wid = lax.axis_index("s") * _NC + lax.axis_index("c")
    base = wid * _BPW

    lanes = lax.iota(jnp.int32, _L)
    masks = {s: (lanes & s) == 0 for s in (8, 4, 2, 1)}
    perms = {s: lanes ^ s for s in (8, 4, 2, 1)}
    bitrev = (((lanes & 1) << 3) | ((lanes & 2) << 1)
              | ((lanes & 4) >> 1) | ((lanes & 8) >> 3))

    def swap(x, s):
        return x.at[perms[s]].get(mode="promise_in_bounds")

    def combine(a, b, s):
        return (jnp.where(masks[s], a, swap(b, s))
                + jnp.where(masks[s], swap(a, s), b))

    def prep_fire(c):
        buf = c % 2
        pltpu.sync_copy(uid_hbm.at[pl.ds(base + c * _CH, _CH)], idu_v.at[c])
        pltpu.sync_copy(iid_hbm.at[pl.ds(base + c * _CH, _CH)], idi_v.at[c])
        for t in range(_CH // _L):
            sl = pl.ds(t * _L, _L)
            ju_v[c, sl] = lax.shift_right_logical(idu_v[c, sl], 1)
            ji_v[c, sl] = lax.shift_right_logical(idi_v[c, sl], 1)
        pltpu.async_copy(ut_hbm.at[ju_v.at[c]], pu_v.at[buf], sem)
        pltpu.async_copy(it_hbm.at[ji_v.at[c]], pi_v.at[buf], sem)

    def drain(c):
        buf = c % 2
        pltpu.make_async_copy(ut_hbm.at[pl.ds(0, _CH)], pu_v.at[buf],
                              sem).wait()
        pltpu.make_async_copy(it_hbm.at[pl.ds(0, _CH)], pi_v.at[buf],
                              sem).wait()

    def compute(c):
        buf = c % 2

        def group(g, carry):
            colv = g * _L
            paru = idu_v[c, pl.ds(colv, _L)] & 1
            pari = idi_v[c, pl.ds(colv, _L)] & 1
            vecs = []
            for j in range(_L):
                r = colv + j
                ou = paru[j] * _D
                oi = pari[j] * _D
                p = (pu_v[buf, r, pl.ds(ou, _L)]
                     * pi_v[buf, r, pl.ds(oi, _L)])
                for q in range(1, _D // _L):
                    p = p + (pu_v[buf, r, pl.ds(ou + q * _L, _L)]
                             * pi_v[buf, r, pl.ds(oi + q * _L, _L)])
                vecs.append(p)
            for s in (8, 4, 2, 1):
                vecs = [combine(vecs[2 * i], vecs[2 * i + 1], s)
                        for i in range(len(vecs) // 2)]
            o_v[pl.ds(c * _CH + colv, _L)] = vecs[0].at[bitrev].get(
                mode="promise_in_bounds")
            return carry

        lax.fori_loop(0, _CH // _L, group, 0)

    prep_fire(0)
    for c in range(_NCH):
        if c + 1 < _NCH:
            prep_fire(c + 1)
        drain(c)
        compute(c)

    pltpu.sync_copy(o_v, out_hbm.at[pl.ds(base, _BPW)])


@jax.jit
def kernel(user_ids, item_ids, user_table, item_table):
    run = functools.partial(
        pl.kernel,
        mesh=plsc.VectorSubcoreMesh(core_axis_name="c", subcore_axis_name="s"),
        out_type=jax.ShapeDtypeStruct((_B,), jnp.float32),
        scratch_types=[
            pltpu.VMEM((_NCH, _CH), jnp.int32),
            pltpu.VMEM((_NCH, _CH), jnp.int32),
            pltpu.VMEM((_NCH, _CH), jnp.int32),
            pltpu.VMEM((_NCH, _CH), jnp.int32),
            pltpu.VMEM((2, _CH, 2 * _D), jnp.float32),
            pltpu.VMEM((2, _CH, 2 * _D), jnp.float32),
            pltpu.VMEM((_BPW,), jnp.float32),
            pltpu.SemaphoreType.DMA,
        ],
    )(_dot_kernel)
    ut2 = user_table.reshape(_PAIRS, 2 * _D)
    it2 = item_table.reshape(_PAIRS, 2 * _D)
    out = run(user_ids, item_ids, ut2, it2)
    return out.reshape(_B, 1)


# speedup vs baseline: 20.6740x; 2.2630x over previous
import functools

import jax
import jax.numpy as jnp
from jax import lax
from jax.experimental import pallas as pl
from jax.experimental.pallas import tpu as pltpu
from jax.experimental.pallas import tpu_sc as plsc

_B = 16384
_D = 64
_NC = 2
_NS = 16
_NW = _NC * _NS
_BPW = _B // _NW
_L = 16
_CH = 16
_NCH = _BPW // _CH
_TILES = 125000


def _dot_kernel(uid_hbm, iid_hbm, ut_hbm, it_hbm, out_hbm,
                idu_v, idi_v, ju_v, ji_v, pu_v, pi_v, o_v, sem):
    wid = lax.axis_index("s") * _NC + lax.axis_index("c")
    base = wid * _BPW

    lanes = lax.iota(jnp.int32, _L)
    masks = {s: (lanes & s) == 0 for s in (8, 4, 2, 1)}
    perms = {s: lanes ^ s for s in (8, 4, 2, 1)}
    bitrev = (((lanes & 1) << 3) | ((lanes & 2) << 1)
              | ((lanes & 4) >> 1) | ((lanes & 8) >> 3))

    def swap(x, s):
        return x.at[perms[s]].get(mode="promise_in_bounds")

    def combine(a, b, s):
        return (jnp.where(masks[s], a, swap(b, s))
                + jnp.where(masks[s], swap(a, s), b))

    pltpu.sync_copy(uid_hbm.at[pl.ds(base, _BPW)], idu_v)
    pltpu.sync_copy(iid_hbm.at[pl.ds(base, _BPW)], idi_v)
    for c in range(_NCH):
        sl = pl.ds(c * _CH, _CH)
        ju_v[c, :] = lax.shift_right_logical(idu_v[sl], 3)
        ji_v[c, :] = lax.shift_right_logical(idi_v[sl], 3)

    def fire(c):
        buf = lax.rem(c, 2)
        juc = ju_v[c, :]
        jic = ji_v[c, :]
        for j in range(_CH):
            pltpu.async_copy(ut_hbm.at[juc[j]], pu_v.at[buf, j], sem)
            pltpu.async_copy(it_hbm.at[jic[j]], pi_v.at[buf, j], sem)

    def drain(c):
        buf = lax.rem(c, 2)
        pltpu.make_async_copy(ut_hbm.at[pl.ds(0, _CH)], pu_v.at[buf],
                              sem).wait()
        pltpu.make_async_copy(it_hbm.at[pl.ds(0, _CH)], pi_v.at[buf],
                              sem).wait()

    def compute(c, buf):
        colv = c * _CH
        subu = idu_v[pl.ds(colv, _L)] & 7
        subi = idi_v[pl.ds(colv, _L)] & 7
        vecs = []
        for j in range(_L):
            bu = subu[j]
            bi = subi[j]
            p = (pu_v[buf, j, bu, pl.ds(0, _L)]
                 * pi_v[buf, j, bi, pl.ds(0, _L)])
            for q in range(1, _D // _L):
                p = p + (pu_v[buf, j, bu, pl.ds(q * _L, _L)]
                         * pi_v[buf, j, bi, pl.ds(q * _L, _L)])
            vecs.append(p)
        for s in (8, 4, 2, 1):
            vecs = [combine(vecs[2 * i], vecs[2 * i + 1], s)
                    for i in range(len(vecs) // 2)]
        o_v[pl.ds(colv, _L)] = vecs[0].at[bitrev].get(
            mode="promise_in_bounds")

    fire(0)

    def step(c, carry):
        @pl.when(c + 1 < _NCH)
        def _():
            fire(c + 1)
        drain(c)
        compute(c, lax.rem(c, 2))
        return carry

    lax.fori_loop(0, _NCH, step, 0)

    pltpu.sync_copy(o_v, out_hbm.at[pl.ds(base, _BPW)])


@jax.jit
def kernel(user_ids, item_ids, user_table, item_table):
    run = functools.partial(
        pl.kernel,
        mesh=plsc.VectorSubcoreMesh(core_axis_name="c", subcore_axis_name="s"),
        out_type=jax.ShapeDtypeStruct((_B,), jnp.float32),
        scratch_types=[
            pltpu.VMEM((_BPW,), jnp.int32),
            pltpu.VMEM((_BPW,), jnp.int32),
            pltpu.VMEM((_NCH, _CH), jnp.int32),
            pltpu.VMEM((_NCH, _CH), jnp.int32),
            pltpu.VMEM((2, _CH, 8, _D), jnp.float32),
            pltpu.VMEM((2, _CH, 8, _D), jnp.float32),
            pltpu.VMEM((_BPW,), jnp.float32),
            pltpu.SemaphoreType.DMA,
        ],
    )(_dot_kernel)
    ut3 = user_table.reshape(_TILES, 8, _D)
    it3 = item_table.reshape(_TILES, 8, _D)
    out = run(user_ids, item_ids, ut3, it3)
    return out.reshape(_B, 1)
